# Initial kernel scaffold; baseline (speedup 1.0000x reference)
import jax
import jax.numpy as jnp
from jax.experimental import pallas as pl


def kernel(X_all, U, z_mu_local):
    raise NotImplementedError("write your pallas kernel here")


# speedup vs baseline: 1.8112x; 1.8112x over previous
import jax
import jax.numpy as jnp
from jax.experimental import pallas as pl

GAMMA = 0.01
ALPHA = 1.0

ROW_BLOCK = 10000


def _body(x_ref, u_ref, zmu_ref, o_ref):
    o_ref[...] = x_ref[...]

    @pl.when(pl.program_id(0) == 0)
    def _compute():
        k = u_ref.shape[1]
        x = x_ref[:k, :]
        zmu = zmu_ref[...]
        diff = x - zmu
        kern = ALPHA * jnp.exp(-GAMMA * jnp.sum(diff * diff, axis=1,
                                                keepdims=True))
        u = u_ref[...]
        proj = jnp.dot(jnp.dot(diff, u, preferred_element_type=jnp.float32),
                       u.T, preferred_element_type=jnp.float32) + zmu
        o_ref[:k, :] = proj * kern + x * (1.0 - kern)


def kernel(X_all, U, z_mu_local):
    n, d = X_all.shape
    k = U.shape[1]
    grid = (n // ROW_BLOCK,)
    return pl.pallas_call(
        _body,
        grid=grid,
        in_specs=[
            pl.BlockSpec((ROW_BLOCK, d), lambda i: (i, 0)),
            pl.BlockSpec((d, k), lambda i: (0, 0)),
            pl.BlockSpec((1, d), lambda i: (0, 0)),
        ],
        out_specs=pl.BlockSpec((ROW_BLOCK, d), lambda i: (i, 0)),
        out_shape=jax.ShapeDtypeStruct((n, d), X_all.dtype),
    )(X_all, U, z_mu_local)


# speedup vs baseline: 1.8507x; 1.0218x over previous
import jax
import jax.numpy as jnp
from jax.experimental import pallas as pl

GAMMA = 0.01
ALPHA = 1.0

ROW_BLOCK = 25000


def _body(x_ref, u_ref, zmu_ref, o_ref):
    o_ref[...] = x_ref[...]

    @pl.when(pl.program_id(0) == 0)
    def _compute():
        k = u_ref.shape[1]
        x = x_ref[:k, :]
        zmu = zmu_ref[...]
        diff = x - zmu
        kern = ALPHA * jnp.exp(-GAMMA * jnp.sum(diff * diff, axis=1,
                                                keepdims=True))
        u = u_ref[...]
        proj = jnp.dot(jnp.dot(diff, u, preferred_element_type=jnp.float32),
                       u.T, preferred_element_type=jnp.float32) + zmu
        o_ref[:k, :] = proj * kern + x * (1.0 - kern)


def kernel(X_all, U, z_mu_local):
    n, d = X_all.shape
    k = U.shape[1]
    grid = (n // ROW_BLOCK,)
    return pl.pallas_call(
        _body,
        grid=grid,
        in_specs=[
            pl.BlockSpec((ROW_BLOCK, d), lambda i: (i, 0)),
            pl.BlockSpec((d, k), lambda i: (0, 0)),
            pl.BlockSpec((1, d), lambda i: (0, 0)),
        ],
        out_specs=pl.BlockSpec((ROW_BLOCK, d), lambda i: (i, 0)),
        out_shape=jax.ShapeDtypeStruct((n, d), X_all.dtype),
    )(X_all, U, z_mu_local)
